# Initial kernel scaffold; baseline (speedup 1.0000x reference)
import jax
import jax.numpy as jnp
from jax.experimental import pallas as pl


def kernel(data):
    raise NotImplementedError("write your pallas kernel here")


# speedup vs baseline: 17.9248x; 17.9248x over previous
import functools

import jax
import jax.numpy as jnp
from jax import lax
from jax.experimental import pallas as pl
from jax.experimental.pallas import tpu as pltpu
from jax.experimental.pallas import tpu_sc as plsc

S = 64
D = 9
C = 4
B = 16
N = 16384
HALVES = 2
XH = S // HALVES
CELLS = XH * S
SENT = CELLS
CNT_W = 2064
OUT_W = CELLS * D * C
OUT_PAD = OUT_W + 48
CHUNK = 4096
NCHUNKS = N // CHUNK
GROUPS = CHUNK // 16

_mesh = functools.partial(
    plsc.VectorSubcoreMesh, core_axis_name="c", subcore_axis_name="s"
)


def _sc_body(data_hbm, out_hbm, buf_v, out_v, cnt_v, in_sem):
  h = lax.axis_index("c")
  b = lax.axis_index("s")

  zf = jnp.zeros((16,), jnp.float32)
  zi = jnp.zeros((16,), jnp.int32)

  def zero_out(i, carry):
    out_v[pl.ds(i * 16, 16)] = zf
    return carry

  lax.fori_loop(0, OUT_PAD // 16, zero_out, 0, unroll=8)

  def zero_cnt(i, carry):
    cnt_v[pl.ds(i * 16, 16)] = zi
    return carry

  lax.fori_loop(0, CNT_W // 16, zero_cnt, 0, unroll=8)

  def chunk_copy(ci, slot):
    return pltpu.make_async_copy(
        data_hbm.at[b, :, pl.ds(ci * CHUNK, CHUNK)], buf_v.at[slot], in_sem
    )

  chunk_copy(0, 0).start()

  hbase = h * CELLS

  for ci in range(NCHUNKS):
    slot = ci % 2
    chunk_copy(ci, slot).wait()
    if ci + 1 < NCHUNKS:
      chunk_copy(ci + 1, 1 - slot).start()
    buf = buf_v.at[slot]

    def group(g, carry):
      off = g * 16
      r0 = buf[0, pl.ds(off, 16)]
      r1 = buf[1, pl.ds(off, 16)]
      r2 = buf[2, pl.ds(off, 16)]
      r3 = buf[3, pl.ds(off, 16)]
      d0 = r0 * float(S)
      d1 = r1 * float(S)
      xi = d0.astype(jnp.int32)
      yi = d1.astype(jnp.int32)
      f0 = d0 - xi.astype(jnp.float32)
      f1 = d1 - yi.astype(jnp.float32)
      nz = (r0 != 0.0) | (r1 != 0.0) | (r2 != 0.0) | (r3 != 0.0)
      mine = (xi >> 5) == h
      lcell = xi * S + yi - hbase
      ceff = jnp.where(nz & mine, lcell, SENT)
      cnt = plsc.load_gather(cnt_v, [ceff])
      dup, last = plsc.scan_count(ceff)
      rank = cnt + dup - 1
      plsc.store_scatter(cnt_v, [ceff], rank + 1, mask=last)
      ok = (ceff != SENT) & (rank < D)
      rc = jnp.minimum(rank, D - 1)
      base = ceff * (D * C) + rc * C
      plsc.store_scatter(out_v, [base], f0, mask=ok)
      plsc.store_scatter(out_v, [base + 1], f1, mask=ok)
      plsc.store_scatter(out_v, [base + 2], r2, mask=ok)
      plsc.store_scatter(out_v, [base + 3], r3, mask=ok)
      return carry

    lax.fori_loop(0, GROUPS, group, 0)

  pltpu.sync_copy(out_v.at[pl.ds(0, OUT_W)], out_hbm.at[b, h])


@jax.jit
def kernel(data):
  launch = pl.kernel(
      _sc_body,
      out_type=jax.ShapeDtypeStruct((B, HALVES, OUT_W), jnp.float32),
      mesh=_mesh(),
      scratch_types=[
          pltpu.VMEM((2, C, CHUNK), jnp.float32),
          pltpu.VMEM((OUT_PAD,), jnp.float32),
          pltpu.VMEM((CNT_W,), jnp.int32),
          pltpu.SemaphoreType.DMA,
      ],
      compiler_params=pltpu.CompilerParams(needs_layout_passes=False),
  )
  dt = jnp.transpose(data, (0, 2, 1))
  out = launch(dt)
  return out.reshape(B, S, S, D * C)
